# baseline (speedup 1.0000x reference)
import jax
import jax.numpy as jnp
from jax import lax
from jax.experimental import pallas as pl
from jax.experimental.pallas import tpu as pltpu
from jax.experimental.pallas import tpu_sc as plsc

U, S, F, D = 33, 256, 64, 64
L = 16
NJ = F // L
NC, NS = 2, 16
NW = NC * NS
SF = S * F
TD, TS = 8, 128
NTR = D // TD
NQ = 4
QSF = SF // NQ
UNIT = TD * QSF
TASKS = U * NTR * NQ
TPW = TASKS // NW
UPT = NTR * NQ
DF = D * F

WV0, BV0, UN0, FL0, FT0 = 0, D, 2 * D, 4 * D, 6 * D
TBL = FT0 + DF


def _sc_body(mf_hbm, tbl_hbm, out_hbm,
             m_v, tbl_v, bu_v, dfl_v, baset_v,
             obuf0, obuf1, sem0, sem1):
    w = lax.axis_index("s") * NC + lax.axis_index("c")
    u0 = (w * TPW) // UPT

    pltpu.sync_copy(mf_hbm.at[pl.ds(u0 * SF, 2 * SF)], m_v)
    pltpu.sync_copy(tbl_hbm, tbl_v)

    zero16 = jnp.zeros((L,), jnp.float32)
    iota16 = lax.iota(jnp.int32, L)

    for j in range(NJ):
        dfl_v[pl.ds(L * j, L)] = (tbl_v[pl.ds(FL0 + D + L * j, L)]
                                  - tbl_v[pl.ds(FL0 + L * j, L)])

    def build_base(u, ui):
        upred = jnp.full((L,), u > 0)
        for j in range(NJ):
            dsj = pl.ds(L * j, L)
            uv = jnp.where(upred, tbl_v[pl.ds(UN0 + D + L * j, L)],
                           tbl_v[pl.ds(UN0 + L * j, L)])
            bu_v[dsj] = (tbl_v[pl.ds(BV0 + L * j, L)] + uv
                         + tbl_v[pl.ds(FL0 + L * j, L)])

        @plsc.parallel_loop(0, D, step=1, unroll=2)
        def dbody(d):
            bub = plsc.load_gather(bu_v, [jnp.full((L,), d, jnp.int32)])
            dfb = plsc.load_gather(dfl_v, [jnp.full((L,), d, jnp.int32)])
            for j in range(NJ):
                fmask = (iota16 + (L * j)) == (u - 1)
                baset_v[pl.ds(ui * DF + d * F + L * j, L)] = (
                    tbl_v[pl.ds(FT0 + d * F + L * j, L)] + bub
                    + jnp.where(fmask, dfb, zero16))

    build_base(u0, 0)
    build_base(u0 + 1, 1)

    def fill_unit(ui, tr, q, buf):
        def ddbody(dd, c):
            d = tr * TD + dd
            wvb = plsc.load_gather(tbl_v, [jnp.full((L,), WV0 + d,
                                                    jnp.int32)])
            bt = [baset_v[pl.ds(ui * DF + d * F + L * j, L)]
                  for j in range(NJ)]

            @plsc.parallel_loop(0, QSF // F, step=1, unroll=16)
            def ibody(i4):
                for jj in range(NJ):
                    moff = ui * SF + q * QSF + i4 * F + L * jj
                    pos = i4 * NJ + jj
                    boff = ((pos // 8) * (TD * TS) + dd * TS
                            + (pos % 8) * L)
                    buf[pl.ds(boff, L)] = (
                        m_v[pl.ds(moff, L)] * wvb + bt[jj])
            return c

        lax.fori_loop(0, TD, ddbody, 0)

    def do_unit(ti, buf, sem, wait_first):
        t = w * TPW + ti
        u = t // UPT
        rem = t % UPT
        tr = rem // NQ
        q = rem % NQ

        if wait_first:
            t2 = t - 2
            pltpu.make_async_copy(
                buf, out_hbm.at[pl.ds(t2 * UNIT, UNIT)], sem).wait()
        fill_unit(u - u0, tr, q, buf)
        pltpu.async_copy(buf, out_hbm.at[pl.ds(t * UNIT, UNIT)], sem)

    do_unit(0, obuf0, sem0, False)
    do_unit(1, obuf1, sem1, False)

    def pair_body(p, c):
        do_unit(2 * p, obuf0, sem0, True)
        do_unit(2 * p + 1, obuf1, sem1, True)
        return c

    lax.fori_loop(1, TPW // 2, pair_body, 0)
    do_unit(TPW - 1, obuf0, sem0, True)

    t_a = w * TPW + TPW - 2
    pltpu.make_async_copy(
        obuf1, out_hbm.at[pl.ds(t_a * UNIT, UNIT)], sem1).wait()
    t_b = w * TPW + TPW - 1
    pltpu.make_async_copy(
        obuf0, out_hbm.at[pl.ds(t_b * UNIT, UNIT)], sem0).wait()


@jax.jit
def _sc_embed(mf, tbl):
    mesh = plsc.VectorSubcoreMesh(
        core_axis_name="c", subcore_axis_name="s",
        num_cores=NC, num_subcores=NS)
    run = pl.kernel(
        _sc_body,
        out_type=jax.ShapeDtypeStruct((TASKS * UNIT,), jnp.float32),
        mesh=mesh,
        compiler_params=pltpu.CompilerParams(needs_layout_passes=False),
        scratch_types=[
            pltpu.VMEM((2 * SF,), jnp.float32),
            pltpu.VMEM((TBL,), jnp.float32),
            pltpu.VMEM((D,), jnp.float32),
            pltpu.VMEM((D,), jnp.float32),
            pltpu.VMEM((2 * DF,), jnp.float32),
            pltpu.VMEM((UNIT,), jnp.float32),
            pltpu.VMEM((UNIT,), jnp.float32),
            pltpu.SemaphoreType.DMA,
            pltpu.SemaphoreType.DMA,
        ],
    )
    return run(mf, tbl)


def kernel(m_data, Wv, bv, feat_table, univ_table, flag_table):
    mf = m_data.reshape(U * S * F)
    tbl = jnp.concatenate([
        Wv.reshape(D), bv, univ_table.reshape(2 * D),
        flag_table.reshape(2 * D), feat_table.T.reshape(DF)])
    out = _sc_embed(mf, tbl)
    out = (out.reshape(U, NTR, SF // TS, TD, TS)
           .transpose(0, 1, 3, 2, 4)
           .reshape(U, D, SF)
           .transpose(0, 2, 1))
    return out


# speedup vs baseline: 1.0419x; 1.0419x over previous
import jax
import jax.numpy as jnp
from jax import lax
from jax.experimental import pallas as pl
from jax.experimental.pallas import tpu as pltpu
from jax.experimental.pallas import tpu_sc as plsc

U, S, F, D = 33, 256, 64, 64
L = 16
NJ = F // L
NC, NS = 2, 16
NW = NC * NS
SF = S * F
TD, TS = 8, 128
NTR = D // TD
NQ = 4
QSF = SF // NQ
UNIT = TD * QSF
TASKS = U * NTR * NQ
TPW = TASKS // NW
UPT = NTR * NQ
DF = D * F

WV0, BV0, UN0, FL0, FT0 = 0, D, 2 * D, 4 * D, 6 * D
TBL = FT0 + DF


def _sc_body(mf_hbm, tbl_hbm, out_hbm,
             m_v, tbl_v, bu_v, dfl_v, baset_v,
             obuf0, obuf1, sem0, sem1):
    w = lax.axis_index("s") * NC + lax.axis_index("c")
    u0 = (w * TPW) // UPT

    pltpu.sync_copy(mf_hbm.at[pl.ds(u0 * SF, 2 * SF)], m_v)
    pltpu.sync_copy(tbl_hbm, tbl_v)

    zero16 = jnp.zeros((L,), jnp.float32)
    iota16 = lax.iota(jnp.int32, L)

    for j in range(NJ):
        dfl_v[pl.ds(L * j, L)] = (tbl_v[pl.ds(FL0 + D + L * j, L)]
                                  - tbl_v[pl.ds(FL0 + L * j, L)])

    def build_base(u, ui):
        upred = jnp.full((L,), u > 0)
        for j in range(NJ):
            dsj = pl.ds(L * j, L)
            uv = jnp.where(upred, tbl_v[pl.ds(UN0 + D + L * j, L)],
                           tbl_v[pl.ds(UN0 + L * j, L)])
            bu_v[dsj] = (tbl_v[pl.ds(BV0 + L * j, L)] + uv
                         + tbl_v[pl.ds(FL0 + L * j, L)])

        @plsc.parallel_loop(0, D, step=1, unroll=2)
        def dbody(d):
            bub = plsc.load_gather(bu_v, [jnp.full((L,), d, jnp.int32)])
            dfb = plsc.load_gather(dfl_v, [jnp.full((L,), d, jnp.int32)])
            for j in range(NJ):
                fmask = (iota16 + (L * j)) == (u - 1)
                baset_v[pl.ds(ui * DF + d * F + L * j, L)] = (
                    tbl_v[pl.ds(FT0 + d * F + L * j, L)] + bub
                    + jnp.where(fmask, dfb, zero16))

    build_base(u0, 0)
    build_base(u0 + 1, 1)

    def fill_unit(ui, tr, q, buf):
        def ddbody(dd, c):
            d = tr * TD + dd
            wvb = plsc.load_gather(tbl_v, [jnp.full((L,), WV0 + d,
                                                    jnp.int32)])
            bt = [baset_v[pl.ds(ui * DF + d * F + L * j, L)]
                  for j in range(NJ)]

            @plsc.parallel_loop(0, QSF // F, step=1, unroll=8)
            def ibody(i4):
                for jj in range(NJ):
                    moff = ui * SF + q * QSF + i4 * F + L * jj
                    pos = i4 * NJ + jj
                    boff = ((pos // 8) * (TD * TS) + dd * TS
                            + (pos % 8) * L)
                    buf[pl.ds(boff, L)] = (
                        m_v[pl.ds(moff, L)] * wvb + bt[jj])
            return c

        lax.fori_loop(0, TD, ddbody, 0)

    def do_unit(ti, buf, sem, wait_first):
        t = w * TPW + ti
        u = t // UPT
        rem = t % UPT
        tr = rem // NQ
        q = rem % NQ

        if wait_first:
            t2 = t - 2
            pltpu.make_async_copy(
                buf, out_hbm.at[pl.ds(t2 * UNIT, UNIT)], sem).wait()
        fill_unit(u - u0, tr, q, buf)
        pltpu.async_copy(buf, out_hbm.at[pl.ds(t * UNIT, UNIT)], sem)

    do_unit(0, obuf0, sem0, False)
    do_unit(1, obuf1, sem1, False)

    def pair_body(p, c):
        do_unit(2 * p, obuf0, sem0, True)
        do_unit(2 * p + 1, obuf1, sem1, True)
        return c

    lax.fori_loop(1, TPW // 2, pair_body, 0)
    do_unit(TPW - 1, obuf0, sem0, True)

    t_a = w * TPW + TPW - 2
    pltpu.make_async_copy(
        obuf1, out_hbm.at[pl.ds(t_a * UNIT, UNIT)], sem1).wait()
    t_b = w * TPW + TPW - 1
    pltpu.make_async_copy(
        obuf0, out_hbm.at[pl.ds(t_b * UNIT, UNIT)], sem0).wait()


@jax.jit
def _sc_embed(mf, tbl):
    mesh = plsc.VectorSubcoreMesh(
        core_axis_name="c", subcore_axis_name="s",
        num_cores=NC, num_subcores=NS)
    run = pl.kernel(
        _sc_body,
        out_type=jax.ShapeDtypeStruct((TASKS * UNIT,), jnp.float32),
        mesh=mesh,
        compiler_params=pltpu.CompilerParams(needs_layout_passes=False),
        scratch_types=[
            pltpu.VMEM((2 * SF,), jnp.float32),
            pltpu.VMEM((TBL,), jnp.float32),
            pltpu.VMEM((D,), jnp.float32),
            pltpu.VMEM((D,), jnp.float32),
            pltpu.VMEM((2 * DF,), jnp.float32),
            pltpu.VMEM((UNIT,), jnp.float32),
            pltpu.VMEM((UNIT,), jnp.float32),
            pltpu.SemaphoreType.DMA,
            pltpu.SemaphoreType.DMA,
        ],
    )
    return run(mf, tbl)


def kernel(m_data, Wv, bv, feat_table, univ_table, flag_table):
    mf = m_data.reshape(U * S * F)
    tbl = jnp.concatenate([
        Wv.reshape(D), bv, univ_table.reshape(2 * D),
        flag_table.reshape(2 * D), feat_table.T.reshape(DF)])
    out = _sc_embed(mf, tbl)
    out = (out.reshape(U, NTR, SF // TS, TD, TS)
           .transpose(0, 1, 3, 2, 4)
           .reshape(U, D, SF)
           .transpose(0, 2, 1))
    return out
